# baseline (speedup 1.0000x reference)
import functools

import jax
import jax.numpy as jnp
from jax import lax
from jax.experimental import pallas as pl
from jax.experimental.pallas import tpu as pltpu
from jax.experimental.pallas import tpu_sc as plsc

_NC, _NS, _L = 2, 16, 16
_NW = _NC * _NS

def _sc_mesh():
    return plsc.VectorSubcoreMesh(
        core_axis_name="c", subcore_axis_name="s",
        num_cores=_NC, num_subcores=_NS)


@functools.lru_cache(maxsize=None)
def _make_deg(ncnt, e):
    ew = e // _NW

    @functools.partial(
        pl.kernel,
        out_type=jax.ShapeDtypeStruct((_NW, 1, ncnt), jnp.float32),
        mesh=_sc_mesh(),
        scratch_types=[
            pltpu.VMEM((ew,), jnp.int32),
            pltpu.VMEM((ncnt,), jnp.float32),
        ],
        compiler_params=pltpu.CompilerParams(needs_layout_passes=False),
    )
    def deg_k(dst_hbm, out_hbm, idx_v, counts_v):
        c = lax.axis_index("c")
        s = lax.axis_index("s")
        wid = s * _NC + c
        zeros = jnp.zeros((_L,), jnp.float32)

        def zero_body(i, carry):
            counts_v[pl.ds(i * _L, _L)] = zeros
            return carry

        lax.fori_loop(0, ncnt // _L, zero_body, 0)
        pltpu.sync_copy(dst_hbm.at[pl.ds(wid * ew, ew)], idx_v)
        ones = jnp.full((_L,), 1.0, jnp.float32)

        def count_body(i, carry):
            iv = idx_v[pl.ds(i * _L, _L)]
            plsc.addupdate_scatter(counts_v, [iv], ones)
            return carry

        lax.fori_loop(0, ew // _L, count_body, 0)
        pltpu.sync_copy(counts_v, out_hbm.at[wid, 0])

    return deg_k


@functools.lru_cache(maxsize=None)
def _pad_rows(n):
    return -(-n // (_NS * 8)) * 8


_K = 128
_NBUF = 2
_NIB = 4


@functools.lru_cache(maxsize=None)
def _make_agg(n, d, e):
    nct = e // _K
    ncw = nct // _NW
    nrem = nct - ncw * _NW
    nps = _pad_rows(n)
    np_tot = nps * _NS

    @functools.partial(
        pl.kernel,
        out_type=jax.ShapeDtypeStruct((_NC, np_tot, d), jnp.float32),
        mesh=_sc_mesh(),
        scratch_types=[
            pltpu.VMEM_SHARED((np_tot, d), jnp.float32),
            pltpu.VMEM((_NIB, 2, _K), jnp.int32),
            pltpu.VMEM((_NBUF, _K, d), jnp.float32),
            pltpu.SemaphoreType.DMA((_NIB,)),
            pltpu.SemaphoreType.DMA((_NBUF,)),
            pltpu.SemaphoreType.DMA((_NBUF,)),
            pltpu.SemaphoreType.DMA,
        ],
    )
    def agg_k(table_hbm, edge_hbm, zeros_hbm, out_hbm,
              acc, ibuf, rows, isem, gsem, ssem, zsem):
        c = lax.axis_index("c")
        s = lax.axis_index("s")
        wid = s * _NC + c
        start = wid * ncw + jnp.minimum(wid, nrem)
        nchw = ncw + jnp.where(wid < nrem, 1, 0)

        zcopy = pltpu.async_copy(zeros_hbm, acc.at[pl.ds(s * nps, nps)], zsem)

        def idx_issue(ch):
            i = lax.rem(ch, _NIB)
            pltpu.async_copy(
                edge_hbm.at[:, pl.ds((start + ch) * _K, _K)],
                ibuf.at[i], isem.at[i])

        def idx_wait(ch):
            i = lax.rem(ch, _NIB)
            pltpu.make_async_copy(
                edge_hbm.at[:, pl.ds((start + ch) * _K, _K)],
                ibuf.at[i], isem.at[i]).wait()

        def gather(ch, b):
            i = lax.rem(ch, _NIB)
            pltpu.async_copy(table_hbm.at[ibuf.at[i, 0]], rows.at[b],
                             gsem.at[b])

        def gather_wait(ch, b):
            i = lax.rem(ch, _NIB)
            pltpu.make_async_copy(table_hbm.at[ibuf.at[i, 0]], rows.at[b],
                                  gsem.at[b]).wait()

        def scatter(ch, b):
            i = lax.rem(ch, _NIB)
            pltpu.async_copy(rows.at[b], acc.at[ibuf.at[i, 1]], ssem.at[b],
                             add=True)

        def scatter_wait(ch, b):
            i = lax.rem(ch, _NIB)
            pltpu.make_async_copy(rows.at[b], acc.at[ibuf.at[i, 1]],
                                  ssem.at[b]).wait()

        for g in range(3):
            idx_issue(g)
        idx_wait(0)
        gather(0, 0)
        zcopy.wait()
        plsc.subcore_barrier()

        def step(ch, carry):
            b = lax.rem(ch, _NBUF)
            bp = lax.rem(ch + 1, _NBUF)
            gather_wait(ch, b)
            scatter(ch, b)

            @pl.when(ch > 0)
            def _():
                scatter_wait(ch - 1, bp)

            @pl.when(ch + 1 < nchw)
            def _():
                idx_wait(ch + 1)
                gather(ch + 1, bp)

            @pl.when(ch + 3 < nchw)
            def _():
                idx_issue(ch + 3)

            return carry

        lax.fori_loop(0, nchw, step, 0)
        scatter_wait(nchw - 1, lax.rem(nchw - 1, _NBUF))
        plsc.subcore_barrier()
        pltpu.sync_copy(acc.at[pl.ds(s * nps, nps)],
                        out_hbm.at[c, pl.ds(s * nps, nps)])

    return agg_k


def _norm_from_counts(cnt_ref, r):
    i = pl.program_id(0)
    cnt = cnt_ref[:, 0, pl.ds(i * r, r)]
    deg = jnp.sum(cnt, axis=0)
    return lax.rsqrt(jnp.maximum(deg, 1.0))


def _mm_scale_body(x_ref, w_ref, cnt_ref, o_ref, *, r):
    nrm = _norm_from_counts(cnt_ref, r)
    h = jnp.dot(x_ref[...], w_ref[...], preferred_element_type=jnp.float32)
    o_ref[...] = h * nrm[:, None]


def _mid_body(aggp_ref, cnt_ref, b_ref, w_ref, o_ref, *, r):
    nrm = _norm_from_counts(cnt_ref, r)
    agg = aggp_ref[0] + aggp_ref[1]
    t = jnp.maximum(agg * nrm[:, None] + b_ref[...], 0.0)
    h = jnp.dot(t, w_ref[...], preferred_element_type=jnp.float32)
    o_ref[...] = h * nrm[:, None]


def _fin_body(aggp_ref, cnt_ref, b_ref, o_ref, *, r):
    nrm = _norm_from_counts(cnt_ref, r)
    agg = aggp_ref[0] + aggp_ref[1]
    o_ref[...] = agg * nrm[:, None] + b_ref[...]


def _tc_calls(n, d, ncnt, r=1280):
    grid = (-(-n // r),)
    row_spec = pl.BlockSpec((r, d), lambda i: (i, 0))
    cnt_spec = pl.BlockSpec((_NW, 1, ncnt), lambda i: (0, 0, 0))
    w_spec = pl.BlockSpec((d, d), lambda i: (0, 0))
    b_spec = pl.BlockSpec((1, d), lambda i: (0, 0))
    agg_spec = pl.BlockSpec((_NC, r, d), lambda i: (0, i, 0))
    out = jax.ShapeDtypeStruct((n, d), jnp.float32)

    mm_scale = pl.pallas_call(
        functools.partial(_mm_scale_body, r=r), grid=grid,
        in_specs=[row_spec, w_spec, cnt_spec],
        out_specs=row_spec, out_shape=out)
    mid = pl.pallas_call(
        functools.partial(_mid_body, r=r), grid=grid,
        in_specs=[agg_spec, cnt_spec, b_spec, w_spec],
        out_specs=row_spec, out_shape=out)
    fin = pl.pallas_call(
        functools.partial(_fin_body, r=r), grid=grid,
        in_specs=[agg_spec, cnt_spec, b_spec],
        out_specs=row_spec, out_shape=out)
    return mm_scale, mid, fin


def kernel(x, edge_index, W0, b0, W1, b1):
    n, d = x.shape
    e = edge_index.shape[1]
    src = edge_index[0]
    dst = edge_index[1]
    b0r = b0.reshape(1, d)
    b1r = b1.reshape(1, d)
    zeros = jnp.zeros((_pad_rows(n), d), jnp.float32)

    ncnt = -(-n // 1280) * 1280
    deg_k = _make_deg(ncnt, e)
    mm_scale, mid, fin = _tc_calls(n, d, ncnt)

    agg_k = _make_agg(n, d, e)

    counts = deg_k(dst)
    h0p = mm_scale(x, W0, counts)
    aggp = agg_k(h0p, edge_index, zeros)
    h1p = mid(aggp, counts, b0r, W1)
    aggp = agg_k(h1p, edge_index, zeros)
    return fin(aggp, counts, b1r)


# speedup vs baseline: 1.1783x; 1.1783x over previous
import functools

import jax
import jax.numpy as jnp
from jax import lax
from jax.experimental import pallas as pl
from jax.experimental.pallas import tpu as pltpu
from jax.experimental.pallas import tpu_sc as plsc

_NC, _NS, _L = 2, 16, 16
_NW = _NC * _NS

def _sc_mesh():
    return plsc.VectorSubcoreMesh(
        core_axis_name="c", subcore_axis_name="s",
        num_cores=_NC, num_subcores=_NS)


@functools.lru_cache(maxsize=None)
def _make_deg(ncnt, e):
    ew = e // _NW

    @functools.partial(
        pl.kernel,
        out_type=jax.ShapeDtypeStruct((_NW, 1, ncnt), jnp.float32),
        mesh=_sc_mesh(),
        scratch_types=[
            pltpu.VMEM((ew,), jnp.int32),
            pltpu.VMEM((ncnt,), jnp.float32),
        ],
        compiler_params=pltpu.CompilerParams(needs_layout_passes=False),
    )
    def deg_k(dst_hbm, out_hbm, idx_v, counts_v):
        c = lax.axis_index("c")
        s = lax.axis_index("s")
        wid = s * _NC + c
        zeros = jnp.zeros((_L,), jnp.float32)

        def zero_body(i, carry):
            counts_v[pl.ds(i * _L, _L)] = zeros
            return carry

        lax.fori_loop(0, ncnt // _L, zero_body, 0)
        pltpu.sync_copy(dst_hbm.at[pl.ds(wid * ew, ew)], idx_v)
        ones = jnp.full((_L,), 1.0, jnp.float32)

        def count_body(i, carry):
            iv = idx_v[pl.ds(i * _L, _L)]
            plsc.addupdate_scatter(counts_v, [iv], ones)
            return carry

        lax.fori_loop(0, ew // _L, count_body, 0)
        pltpu.sync_copy(counts_v, out_hbm.at[wid, 0])

    return deg_k


@functools.lru_cache(maxsize=None)
def _pad_rows(n):
    return -(-n // (_NS * 8)) * 8


_K = 100
_NBUF = 3
_GD = _NBUF - 1
_ID = 2 * _GD
_NIB = _ID + 2


@functools.lru_cache(maxsize=None)
def _make_agg(n, d, nch):
    nps = _pad_rows(n)
    np_tot = nps * _NS

    @functools.partial(
        pl.kernel,
        out_type=jax.ShapeDtypeStruct((_NC, np_tot, d), jnp.float32),
        mesh=_sc_mesh(),
        scratch_types=[
            pltpu.VMEM_SHARED((np_tot, d), jnp.float32),
            pltpu.VMEM((_NIB, 2, _K), jnp.int32),
            pltpu.VMEM((_NBUF, _K, d), jnp.float32),
            pltpu.SemaphoreType.DMA((_NIB,)),
            pltpu.SemaphoreType.DMA((_NBUF,)),
            pltpu.SemaphoreType.DMA((_NBUF,)),
            pltpu.SemaphoreType.DMA,
        ],
    )
    def agg_k(table_hbm, idx_hbm, zeros_hbm, out_hbm,
              acc, ibuf, rows, isem, gsem, ssem, zsem):
        c = lax.axis_index("c")
        s = lax.axis_index("s")
        wid = s * _NC + c

        zcopy = pltpu.async_copy(zeros_hbm, acc.at[pl.ds(s * nps, nps)], zsem)

        def idx_issue(ch):
            i = lax.rem(ch, _NIB)
            pltpu.async_copy(idx_hbm.at[wid, ch], ibuf.at[i], isem.at[i])

        def idx_wait(ch):
            i = lax.rem(ch, _NIB)
            pltpu.make_async_copy(idx_hbm.at[wid, ch], ibuf.at[i],
                                  isem.at[i]).wait()

        def gather(ch, b):
            i = lax.rem(ch, _NIB)
            pltpu.async_copy(table_hbm.at[ibuf.at[i, 0]], rows.at[b],
                             gsem.at[b])

        def gather_wait(ch, b):
            i = lax.rem(ch, _NIB)
            pltpu.make_async_copy(table_hbm.at[ibuf.at[i, 0]], rows.at[b],
                                  gsem.at[b]).wait()

        def scatter(ch, b):
            i = lax.rem(ch, _NIB)
            pltpu.async_copy(rows.at[b], acc.at[ibuf.at[i, 1]], ssem.at[b],
                             add=True)

        def scatter_wait(ch, b):
            i = lax.rem(ch, _NIB)
            pltpu.make_async_copy(rows.at[b], acc.at[ibuf.at[i, 1]],
                                  ssem.at[b]).wait()

        for g in range(min(_ID, nch)):
            idx_issue(g)
        for g in range(min(_GD, nch)):
            idx_wait(g)
            gather(g, g)
        zcopy.wait()
        plsc.subcore_barrier()

        def step(ch, carry):
            b = lax.rem(ch, _NBUF)
            bp = lax.rem(ch + _NBUF - 1, _NBUF)
            gather_wait(ch, b)
            scatter(ch, b)

            @pl.when(ch > 0)
            def _():
                scatter_wait(ch - 1, bp)

            @pl.when(ch + _GD < nch)
            def _():
                idx_wait(ch + _GD)
                gather(ch + _GD, bp)

            @pl.when(ch + _ID < nch)
            def _():
                idx_issue(ch + _ID)

            return carry

        lax.fori_loop(0, nch, step, 0)
        scatter_wait(nch - 1, (nch - 1) % _NBUF)
        plsc.subcore_barrier()
        pltpu.sync_copy(acc.at[pl.ds(s * nps, nps)],
                        out_hbm.at[c, pl.ds(s * nps, nps)])

    return agg_k


def _norm_from_counts(cnt_ref, r):
    i = pl.program_id(0)
    cnt = cnt_ref[:, 0, pl.ds(i * r, r)]
    deg = jnp.sum(cnt, axis=0)
    return lax.rsqrt(jnp.maximum(deg, 1.0))


def _mm_scale_body(x_ref, w_ref, cnt_ref, o_ref, *, r):
    nrm = _norm_from_counts(cnt_ref, r)
    h = jnp.dot(x_ref[...], w_ref[...], preferred_element_type=jnp.float32)
    o_ref[...] = h * nrm[:, None]


def _mid_body(aggp_ref, cnt_ref, b_ref, w_ref, o_ref, *, r):
    nrm = _norm_from_counts(cnt_ref, r)
    agg = aggp_ref[0] + aggp_ref[1]
    t = jnp.maximum(agg * nrm[:, None] + b_ref[...], 0.0)
    h = jnp.dot(t, w_ref[...], preferred_element_type=jnp.float32)
    o_ref[...] = h * nrm[:, None]


def _fin_body(aggp_ref, cnt_ref, b_ref, o_ref, *, r):
    nrm = _norm_from_counts(cnt_ref, r)
    agg = aggp_ref[0] + aggp_ref[1]
    o_ref[...] = agg * nrm[:, None] + b_ref[...]


def _tc_calls(n, d, ncnt, r=1280):
    grid = (-(-n // r),)
    row_spec = pl.BlockSpec((r, d), lambda i: (i, 0))
    cnt_spec = pl.BlockSpec((_NW, 1, ncnt), lambda i: (0, 0, 0))
    w_spec = pl.BlockSpec((d, d), lambda i: (0, 0))
    b_spec = pl.BlockSpec((1, d), lambda i: (0, 0))
    agg_spec = pl.BlockSpec((_NC, r, d), lambda i: (0, i, 0))
    out = jax.ShapeDtypeStruct((n, d), jnp.float32)

    mm_scale = pl.pallas_call(
        functools.partial(_mm_scale_body, r=r), grid=grid,
        in_specs=[row_spec, w_spec, cnt_spec],
        out_specs=row_spec, out_shape=out)
    mid = pl.pallas_call(
        functools.partial(_mid_body, r=r), grid=grid,
        in_specs=[agg_spec, cnt_spec, b_spec, w_spec],
        out_specs=row_spec, out_shape=out)
    fin = pl.pallas_call(
        functools.partial(_fin_body, r=r), grid=grid,
        in_specs=[agg_spec, cnt_spec, b_spec],
        out_specs=row_spec, out_shape=out)
    return mm_scale, mid, fin


def kernel(x, edge_index, W0, b0, W1, b1):
    n, d = x.shape
    e = edge_index.shape[1]
    src = edge_index[0]
    dst = edge_index[1]
    b0r = b0.reshape(1, d)
    b1r = b1.reshape(1, d)
    zeros = jnp.zeros((_pad_rows(n), d), jnp.float32)

    ncnt = -(-n // 1280) * 1280
    deg_k = _make_deg(ncnt, e)
    mm_scale, mid, fin = _tc_calls(n, d, ncnt)

    ew = e // _NW
    ew_pad = -(-ew // _K) * _K
    nch = ew_pad // _K
    np_tot = _pad_rows(n) * _NS
    srcw = src.reshape(_NW, ew)
    dstw = dst.reshape(_NW, ew)
    if ew_pad != ew:
        sinks = n + jnp.arange(_NW, dtype=jnp.int32) % (np_tot - n)
        srcw = jnp.pad(srcw, ((0, 0), (0, ew_pad - ew)))
        dstw = jnp.concatenate(
            [dstw, jnp.broadcast_to(sinks[:, None], (_NW, ew_pad - ew))],
            axis=1)
    idx4 = jnp.stack(
        [srcw.reshape(_NW, nch, _K), dstw.reshape(_NW, nch, _K)], axis=2)
    agg_k = _make_agg(n, d, nch)

    counts = deg_k(dst)
    h0p = mm_scale(x, W0, counts)
    aggp = agg_k(h0p, idx4, zeros)
    h1p = mid(aggp, counts, b0r, W1)
    aggp = agg_k(h1p, idx4, zeros)
    return fin(aggp, counts, b1r)
